# Initial kernel scaffold; baseline (speedup 1.0000x reference)
import jax
import jax.numpy as jnp
from jax.experimental import pallas as pl


def kernel(z, edge_index, edge_type, rel_emb):
    raise NotImplementedError("write your pallas kernel here")


# speedup vs baseline: 4.0464x; 4.0464x over previous
import functools

import jax
import jax.numpy as jnp
from jax import lax
from jax.experimental import pallas as pl
from jax.experimental.pallas import tpu as pltpu
from jax.experimental.pallas import tpu_sc as plsc

E = 320000
H = 128
NC = 2
NS = 16
NW = NC * NS
EPW = E // NW
K = 80
NCHUNK = EPW // K
G = K // 16

_mesh = plsc.VectorSubcoreMesh(core_axis_name="c", subcore_axis_name="s")


@functools.partial(
    pl.kernel,
    mesh=_mesh,
    out_type=jax.ShapeDtypeStruct((E,), jnp.float32),
    compiler_params=pltpu.CompilerParams(needs_layout_passes=False),
    scratch_types=[
        pltpu.VMEM((K,), jnp.int32),
        pltpu.VMEM((K,), jnp.int32),
        pltpu.VMEM((K,), jnp.int32),
        pltpu.VMEM((K, H), jnp.float32),
        pltpu.VMEM((K, H), jnp.float32),
        pltpu.VMEM((K, H), jnp.float32),
        pltpu.VMEM((K,), jnp.float32),
        pltpu.SemaphoreType.DMA,
        pltpu.SemaphoreType.DMA,
        pltpu.SemaphoreType.DMA,
    ],
)
def _distmult_sc(src_hbm, dst_hbm, typ_hbm, z_hbm, rel_hbm, out_hbm,
                 sidx_v, didx_v, tidx_v, zsrc_v, zdst_v, rel_v, out_v,
                 sem_s, sem_d, sem_r):
    wid = lax.axis_index("s") * NC + lax.axis_index("c")
    row16 = lax.iota(jnp.int32, 16)

    def chunk_body(c, carry):
        base = wid * EPW + c * K
        pltpu.sync_copy(src_hbm.at[pl.ds(base, K)], sidx_v)
        pltpu.sync_copy(dst_hbm.at[pl.ds(base, K)], didx_v)
        pltpu.sync_copy(typ_hbm.at[pl.ds(base, K)], tidx_v)
        cs = pltpu.async_copy(z_hbm.at[sidx_v], zsrc_v, sem_s)
        cd = pltpu.async_copy(z_hbm.at[didx_v], zdst_v, sem_d)
        cr = pltpu.async_copy(rel_hbm.at[tidx_v], rel_v, sem_r)
        cs.wait()
        cd.wait()
        cr.wait()

        def group_body(g, carry2):
            def edge_body(e16, acc_out):
                e = g * 16 + e16
                acc = jnp.zeros((16,), jnp.float32)
                for j in range(H // 16):
                    sl = pl.ds(j * 16, 16)
                    acc = acc + zsrc_v[e, sl] * rel_v[e, sl] * zdst_v[e, sl]
                s = jnp.sum(acc)
                return jnp.where(row16 == e16, s, acc_out)

            acc_out = lax.fori_loop(0, 16, edge_body,
                                    jnp.zeros((16,), jnp.float32))
            out_v[pl.ds(g * 16, 16)] = acc_out
            return carry2

        lax.fori_loop(0, G, group_body, 0)
        pltpu.sync_copy(out_v, out_hbm.at[pl.ds(base, K)])
        return carry

    lax.fori_loop(0, NCHUNK, chunk_body, 0)


def kernel(z, edge_index, edge_type, rel_emb):
    src = edge_index[0].astype(jnp.int32)
    dst = edge_index[1].astype(jnp.int32)
    typ = edge_type.astype(jnp.int32)
    return _distmult_sc(src, dst, typ, z, rel_emb)
